# Initial kernel scaffold; baseline (speedup 1.0000x reference)
import jax
import jax.numpy as jnp
from jax.experimental import pallas as pl


def kernel(x, edge_index, W0, b0, W1, b1):
    raise NotImplementedError("write your pallas kernel here")


# speedup vs baseline: 7.5461x; 7.5461x over previous
import functools

import jax
import jax.numpy as jnp
from jax import lax
from jax.experimental import pallas as pl
from jax.experimental.pallas import tpu as pltpu
from jax.experimental.pallas import tpu_sc as plsc

N = 10000
E = 320000
D = 128

NC = 2
NS = 16
NW = NC * NS
EPW = E // NW
CH = 80
NCHUNK = EPW // CH
ZR = 16

_MESH = plsc.VectorSubcoreMesh(core_axis_name="c", subcore_axis_name="s")

_f32 = jnp.float32
_i32 = jnp.int32


def _sc_edge_body(x_hbm, src_hbm, dst_hbm, ew_hbm, degp_hbm,
                  sidx, didx, rows_s, rows_d, psum, ewb, deg, sem1, sem2):
    cid = lax.axis_index("c")
    sid = lax.axis_index("s")
    wid = sid * NC + cid
    ebase = wid * EPW

    zero16 = jnp.zeros((16,), _f32)

    def _zero(i, carry):
        deg[pl.ds(i * 16, 16)] = zero16
        return carry

    lax.fori_loop(0, N // 16, _zero, 0)

    iot = lax.iota(_i32, 16)

    def _chunk(c, carry):
        off = ebase + c * CH
        pltpu.sync_copy(src_hbm.at[pl.ds(off, CH)], sidx)
        pltpu.sync_copy(dst_hbm.at[pl.ds(off, CH)], didx)
        cp1 = pltpu.async_copy(x_hbm.at[sidx], rows_s, sem1)
        cp2 = pltpu.async_copy(x_hbm.at[didx], rows_d, sem2)
        cp1.wait()
        cp2.wait()

        def _edge(e, ecarry):
            s = rows_s[e, pl.ds(0, 16)] * rows_d[e, pl.ds(0, 16)]
            for j in range(1, 8):
                s = s + (rows_s[e, pl.ds(j * 16, 16)] *
                         rows_d[e, pl.ds(j * 16, 16)])
            psum[pl.ds(e * 16, 16)] = s
            return ecarry

        lax.fori_loop(0, CH, _edge, 0)

        def _grp(g, gcarry):
            rowv = (g * 16 + iot) * 16
            acc = plsc.load_gather(psum, [rowv])
            for l in range(1, 16):
                acc = acc + plsc.load_gather(psum, [rowv + l])
            w = 1.0 / (1.0 + jnp.exp(-acc))
            ewb[pl.ds(g * 16, 16)] = w
            d16 = didx[pl.ds(g * 16, 16)]
            plsc.addupdate_scatter(deg, [d16], w)
            return gcarry

        lax.fori_loop(0, CH // 16, _grp, 0)

        pltpu.sync_copy(ewb, ew_hbm.at[pl.ds(off, CH)])
        return carry

    lax.fori_loop(0, NCHUNK, _chunk, 0)

    pltpu.sync_copy(deg, degp_hbm.at[pl.ds(wid * N, N)])


_SC_PARAMS = pltpu.CompilerParams(needs_layout_passes=False)

_sc_edge = pl.kernel(
    _sc_edge_body,
    out_type=[jax.ShapeDtypeStruct((E,), _f32),
              jax.ShapeDtypeStruct((NW * N,), _f32)],
    mesh=_MESH,
    compiler_params=_SC_PARAMS,
    scratch_types=[
        pltpu.VMEM((CH,), _i32),
        pltpu.VMEM((CH,), _i32),
        pltpu.VMEM((CH, D), _f32),
        pltpu.VMEM((CH, D), _f32),
        pltpu.VMEM((CH * 16,), _f32),
        pltpu.VMEM((CH,), _f32),
        pltpu.VMEM((N,), _f32),
        pltpu.SemaphoreType.DMA,
        pltpu.SemaphoreType.DMA,
    ],
)


def _sc_agg_body(h_hbm, src_hbm, dst_hbm, ew_hbm, dis_hbm, part_hbm,
                 sidx, didx, ewb, fbuf, rows, disv, zbuf, acc, sem1):
    cid = lax.axis_index("c")
    sid = lax.axis_index("s")
    wid = sid * NC + cid
    ebase = wid * EPW

    pltpu.sync_copy(dis_hbm, disv)

    rbase = sid * 624
    nblk = 39 + jnp.where(sid == NS - 1, 1, 0)

    zero16 = jnp.zeros((16,), _f32)

    def _zrow(r, carry):
        for k in range(D // 16):
            zbuf[r, pl.ds(k * 16, 16)] = zero16
        return carry

    lax.fori_loop(0, ZR, _zrow, 0)

    def _zfill(t, carry):
        pltpu.sync_copy(zbuf, acc.at[pl.ds(rbase + t * ZR, ZR)])
        return carry

    lax.fori_loop(0, nblk, _zfill, 0)
    plsc.subcore_barrier()

    zero16i = jnp.zeros((16,), _i32)

    def _chunk(c, carry):
        off = ebase + c * CH
        pltpu.sync_copy(src_hbm.at[pl.ds(off, CH)], sidx)
        pltpu.sync_copy(dst_hbm.at[pl.ds(off, CH)], didx)
        pltpu.sync_copy(ew_hbm.at[pl.ds(off, CH)], ewb)
        pltpu.async_copy(h_hbm.at[sidx], rows, sem1).wait()

        def _fact(g, gcarry):
            s16 = sidx[pl.ds(g * 16, 16)]
            w16 = ewb[pl.ds(g * 16, 16)] * plsc.load_gather(disv, [s16])
            fbuf[pl.ds(g * 16, 16)] = w16
            return gcarry

        lax.fori_loop(0, CH // 16, _fact, 0)

        def _edge(e, ecarry):
            w = plsc.load_gather(fbuf, [zero16i + e])
            for j in range(D // 16):
                rows[e, pl.ds(j * 16, 16)] = rows[e, pl.ds(j * 16, 16)] * w
            return ecarry

        lax.fori_loop(0, CH, _edge, 0)

        pltpu.sync_copy(rows, acc.at[didx], add=True)
        return carry

    lax.fori_loop(0, NCHUNK, _chunk, 0)
    plsc.subcore_barrier()

    def _flush(t, carry):
        r0 = rbase + t * ZR
        pltpu.sync_copy(acc.at[pl.ds(r0, ZR)], part_hbm.at[cid, pl.ds(r0, ZR)])
        return carry

    lax.fori_loop(0, nblk, _flush, 0)


_sc_agg = pl.kernel(
    _sc_agg_body,
    out_type=[jax.ShapeDtypeStruct((NC, N, D), _f32)],
    mesh=_MESH,
    compiler_params=_SC_PARAMS,
    scratch_types=[
        pltpu.VMEM((CH,), _i32),
        pltpu.VMEM((CH,), _i32),
        pltpu.VMEM((CH,), _f32),
        pltpu.VMEM((CH,), _f32),
        pltpu.VMEM((CH, D), _f32),
        pltpu.VMEM((N,), _f32),
        pltpu.VMEM((ZR, D), _f32),
        pltpu.VMEM_SHARED((N, D), _f32),
        pltpu.SemaphoreType.DMA,
    ],
)


def _tc_b_body(x_ref, w_ref, degp_ref, h_ref, dis_ref):
    deg = 1.0 + jnp.sum(degp_ref[...], axis=0)
    dis_ref[...] = lax.rsqrt(deg)
    h_ref[...] = lax.dot_general(
        x_ref[...], w_ref[...], (((1,), (0,)), ((), ())),
        precision=lax.Precision.HIGHEST, preferred_element_type=_f32)


def _tc_d_body(part_ref, h_ref, dis_ref, b_ref, w_ref, out_ref):
    dis = dis_ref[...][:, None]
    agg = part_ref[0] + part_ref[1]
    h1 = jnp.maximum(dis * agg + dis * dis * h_ref[...] + b_ref[...][None, :],
                     0.0)
    out_ref[...] = lax.dot_general(
        h1, w_ref[...], (((1,), (0,)), ((), ())),
        precision=lax.Precision.HIGHEST, preferred_element_type=_f32)


def _tc_f_body(part_ref, g_ref, dis_ref, b_ref, out_ref):
    dis = dis_ref[...][:, None]
    agg = part_ref[0] + part_ref[1]
    out_ref[...] = dis * agg + dis * dis * g_ref[...] + b_ref[...][None, :]


_tc_b = pl.pallas_call(
    _tc_b_body,
    out_shape=[jax.ShapeDtypeStruct((N, D), _f32),
               jax.ShapeDtypeStruct((N,), _f32)],
)

_tc_d = pl.pallas_call(
    _tc_d_body,
    out_shape=jax.ShapeDtypeStruct((N, D), _f32),
)

_tc_f = pl.pallas_call(
    _tc_f_body,
    out_shape=jax.ShapeDtypeStruct((N, D), _f32),
)


def kernel(x, edge_index, W0, b0, W1, b1):
    src = edge_index[0]
    dst = edge_index[1]
    ew, degp = _sc_edge(x, src, dst)
    h0, dis = _tc_b(x, W0, degp.reshape(NW, N))
    part0, = _sc_agg(h0, src, dst, ew, dis)
    g1 = _tc_d(part0, h0, dis, b0, W1)
    part1, = _sc_agg(g1, src, dst, ew, dis)
    out = _tc_f(part1, g1, dis, b1)
    return (out, edge_index, ew)


# speedup vs baseline: 19.6005x; 2.5974x over previous
import jax
import jax.numpy as jnp
from jax import lax
from jax.experimental import pallas as pl
from jax.experimental.pallas import tpu as pltpu
from jax.experimental.pallas import tpu_sc as plsc

N = 10000
E = 320000
D = 128

NC = 2
NS = 16
NW = NC * NS
EPW = E // NW
CH = 80
NCHUNK = EPW // CH
ZR = 16

_MESH = plsc.VectorSubcoreMesh(core_axis_name="c", subcore_axis_name="s")
_SC_PARAMS = pltpu.CompilerParams(needs_layout_passes=False)

_f32 = jnp.float32
_i32 = jnp.int32


def _sc_edge_body(x_hbm, src_hbm, dst_hbm, ew_hbm, degp_hbm,
                  sidx_all, didx_all, ew_all, deg, psum,
                  rows_s0, rows_s1, rows_d0, rows_d1,
                  gss0, gss1, gsd0, gsd1):
    cid = lax.axis_index("c")
    sid = lax.axis_index("s")
    wid = sid * NC + cid
    ebase = wid * EPW

    rows_s = (rows_s0, rows_s1)
    rows_d = (rows_d0, rows_d1)
    gss = (gss0, gss1)
    gsd = (gsd0, gsd1)

    pltpu.sync_copy(src_hbm.at[pl.ds(ebase, EPW)], sidx_all)
    pltpu.sync_copy(dst_hbm.at[pl.ds(ebase, EPW)], didx_all)

    zero16 = jnp.zeros((16,), _f32)

    def _zero(i, carry):
        deg[pl.ds(i * 16, 16)] = zero16
        return carry

    lax.fori_loop(0, N // 16, _zero, 0)

    iot = lax.iota(_i32, 16)

    def _gather(c, b):
        pltpu.async_copy(x_hbm.at[sidx_all.at[pl.ds(c * CH, CH)]],
                         rows_s[b], gss[b])
        pltpu.async_copy(x_hbm.at[didx_all.at[pl.ds(c * CH, CH)]],
                         rows_d[b], gsd[b])

    def _wait(b):
        pltpu.make_async_copy(x_hbm.at[pl.ds(0, CH)], rows_s[b],
                              gss[b]).wait()
        pltpu.make_async_copy(x_hbm.at[pl.ds(0, CH)], rows_d[b],
                              gsd[b]).wait()

    def _compute(c, b):
        cb = c * CH

        def _edge(e, ecarry):
            s = rows_s[b][e, pl.ds(0, 16)] * rows_d[b][e, pl.ds(0, 16)]
            for j in range(1, D // 16):
                s = s + (rows_s[b][e, pl.ds(j * 16, 16)] *
                         rows_d[b][e, pl.ds(j * 16, 16)])
            psum[pl.ds(e * 16, 16)] = s
            return ecarry

        lax.fori_loop(0, CH, _edge, 0)

        def _grp(g, gcarry):
            rowv = (g * 16 + iot) * 16
            acc = plsc.load_gather(psum, [rowv])
            for l in range(1, 16):
                acc = acc + plsc.load_gather(psum, [rowv + l])
            w = 1.0 / (1.0 + jnp.exp(-acc))
            ew_all[pl.ds(cb + g * 16, 16)] = w
            d16 = didx_all[pl.ds(cb + g * 16, 16)]
            plsc.addupdate_scatter(deg, [d16], w)
            return gcarry

        lax.fori_loop(0, CH // 16, _grp, 0)

    def _chunk(c, b, bo):
        cg = jnp.minimum(c + 1, NCHUNK - 1)
        _gather(cg, bo)
        _wait(b)
        _compute(c, b)

    _gather(0, 0)
    _chunk(0, 0, 1)

    def _super(s, carry):
        _chunk(2 * s + 1, 1, 0)
        _chunk(2 * s + 2, 0, 1)
        return carry

    lax.fori_loop(0, (NCHUNK - 1) // 2, _super, 0)

    _wait(1)

    pltpu.sync_copy(ew_all, ew_hbm.at[pl.ds(ebase, EPW)])
    pltpu.sync_copy(deg, degp_hbm.at[pl.ds(wid * N, N)])


_sc_edge = pl.kernel(
    _sc_edge_body,
    out_type=[jax.ShapeDtypeStruct((E,), _f32),
              jax.ShapeDtypeStruct((NW * N,), _f32)],
    mesh=_MESH,
    compiler_params=_SC_PARAMS,
    scratch_types=[
        pltpu.VMEM((EPW,), _i32),
        pltpu.VMEM((EPW,), _i32),
        pltpu.VMEM((EPW,), _f32),
        pltpu.VMEM((N,), _f32),
        pltpu.VMEM((CH * 16,), _f32),
        pltpu.VMEM((CH, D), _f32),
        pltpu.VMEM((CH, D), _f32),
        pltpu.VMEM((CH, D), _f32),
        pltpu.VMEM((CH, D), _f32),
        pltpu.SemaphoreType.DMA,
        pltpu.SemaphoreType.DMA,
        pltpu.SemaphoreType.DMA,
        pltpu.SemaphoreType.DMA,
    ],
)


def _sc_agg_body(h_hbm, src_hbm, dst_hbm, ew_hbm, part_hbm,
                 sidx_all,
                 rows0, rows1, rows2, ewb0, ewb1, ewb2,
                 didxb0, didxb1, didxb2, acc,
                 gs0, gs1, gs2, ss0, ss1, ss2):
    cid = lax.axis_index("c")
    sid = lax.axis_index("s")
    wid = sid * NC + cid
    ebase = wid * EPW

    rows = (rows0, rows1, rows2)
    ewb = (ewb0, ewb1, ewb2)
    didxb = (didxb0, didxb1, didxb2)
    gs = (gs0, gs1, gs2)
    ss = (ss0, ss1, ss2)

    pltpu.sync_copy(src_hbm.at[pl.ds(ebase, EPW)], sidx_all)

    rbase = sid * 624
    nblk = 39 + jnp.where(sid == NS - 1, 1, 0)

    zero16 = jnp.zeros((16,), _f32)

    def _zrow(r, carry):
        for k in range(D // 16):
            rows0[r, pl.ds(k * 16, 16)] = zero16
        return carry

    lax.fori_loop(0, ZR, _zrow, 0)

    def _zfill(t, carry):
        pltpu.sync_copy(rows0.at[pl.ds(0, ZR)],
                        acc.at[pl.ds(rbase + t * ZR, ZR)])
        return carry

    lax.fori_loop(0, nblk, _zfill, 0)
    plsc.subcore_barrier()

    def _gather(c, b):
        off = ebase + c * CH
        pltpu.async_copy(h_hbm.at[sidx_all.at[pl.ds(c * CH, CH)]],
                         rows[b], gs[b])
        pltpu.async_copy(ew_hbm.at[pl.ds(off, CH)], ewb[b], gs[b])
        pltpu.async_copy(dst_hbm.at[pl.ds(off, CH)], didxb[b], gs[b])

    def _wait_rows(b):
        pltpu.make_async_copy(h_hbm.at[pl.ds(0, CH)], rows[b], gs[b]).wait()
        pltpu.make_async_copy(ew_hbm.at[pl.ds(0, CH)], ewb[b], gs[b]).wait()
        pltpu.make_async_copy(dst_hbm.at[pl.ds(0, CH)], didxb[b],
                              gs[b]).wait()

    def _wait_scat(b):
        pltpu.make_async_copy(h_hbm.at[pl.ds(0, CH)], rows[b], ss[b]).wait()

    zero16i = jnp.zeros((16,), _i32)

    def _compute(c, b):
        def _edge(e, ecarry):
            w = plsc.load_gather(ewb[b], [zero16i + e])
            for j in range(D // 16):
                rows[b][e, pl.ds(j * 16, 16)] = (
                    rows[b][e, pl.ds(j * 16, 16)] * w)
            return ecarry

        lax.fori_loop(0, CH, _edge, 0)

    def _scatter(c, b):
        pltpu.async_copy(rows[b], acc.at[didxb[b]], ss[b], add=True)

    def _chunk(c, b, bn):
        _wait_scat(bn)
        cg = jnp.minimum(c + 1, NCHUNK - 1)
        _gather(cg, bn)
        _wait_rows(b)
        _compute(c, b)
        _scatter(c, b)

    _gather(0, 0)
    _gather(1, 1)
    _wait_rows(0)
    _compute(0, 0)
    _scatter(0, 0)
    _gather(2, 2)
    _wait_rows(1)
    _compute(1, 1)
    _scatter(1, 1)

    def _super(s, carry):
        _chunk(3 * s + 2, 2, 0)
        _chunk(3 * s + 3, 0, 1)
        _chunk(3 * s + 4, 1, 2)
        return carry

    lax.fori_loop(0, (NCHUNK - 2) // 3, _super, 0)

    _wait_scat(0)
    _wait_scat(1)
    _wait_rows(2)
    plsc.subcore_barrier()

    def _flush(t, carry):
        r0 = rbase + t * ZR
        pltpu.sync_copy(acc.at[pl.ds(r0, ZR)], part_hbm.at[cid, pl.ds(r0, ZR)])
        return carry

    lax.fori_loop(0, nblk, _flush, 0)


_sc_agg = pl.kernel(
    _sc_agg_body,
    out_type=[jax.ShapeDtypeStruct((NC, N, D), _f32)],
    mesh=_MESH,
    compiler_params=_SC_PARAMS,
    scratch_types=[
        pltpu.VMEM((EPW,), _i32),
        pltpu.VMEM((CH, D), _f32),
        pltpu.VMEM((CH, D), _f32),
        pltpu.VMEM((CH, D), _f32),
        pltpu.VMEM((CH,), _f32),
        pltpu.VMEM((CH,), _f32),
        pltpu.VMEM((CH,), _f32),
        pltpu.VMEM((CH,), _i32),
        pltpu.VMEM((CH,), _i32),
        pltpu.VMEM((CH,), _i32),
        pltpu.VMEM_SHARED((N, D), _f32),
        pltpu.SemaphoreType.DMA,
        pltpu.SemaphoreType.DMA,
        pltpu.SemaphoreType.DMA,
        pltpu.SemaphoreType.DMA,
        pltpu.SemaphoreType.DMA,
        pltpu.SemaphoreType.DMA,
    ],
)


def _tc_b_body(x_ref, w_ref, degp_ref, h_ref, dis_ref):
    deg = 1.0 + jnp.sum(degp_ref[...], axis=0)
    dis = lax.rsqrt(deg)
    dis_ref[...] = dis
    h_ref[...] = dis[:, None] * lax.dot_general(
        x_ref[...], w_ref[...], (((1,), (0,)), ((), ())),
        precision=lax.Precision.HIGHEST, preferred_element_type=_f32)


def _tc_d_body(part_ref, h_ref, dis_ref, b_ref, w_ref, out_ref):
    dis = dis_ref[...][:, None]
    agg = part_ref[0] + part_ref[1]
    h1 = jnp.maximum(dis * agg + dis * h_ref[...] + b_ref[...][None, :],
                     0.0)
    out_ref[...] = dis * lax.dot_general(
        h1, w_ref[...], (((1,), (0,)), ((), ())),
        precision=lax.Precision.HIGHEST, preferred_element_type=_f32)


def _tc_f_body(part_ref, g_ref, dis_ref, b_ref, out_ref):
    dis = dis_ref[...][:, None]
    agg = part_ref[0] + part_ref[1]
    out_ref[...] = dis * agg + dis * g_ref[...] + b_ref[...][None, :]


_tc_b = pl.pallas_call(
    _tc_b_body,
    out_shape=[jax.ShapeDtypeStruct((N, D), _f32),
               jax.ShapeDtypeStruct((N,), _f32)],
)

_tc_d = pl.pallas_call(
    _tc_d_body,
    out_shape=jax.ShapeDtypeStruct((N, D), _f32),
)

_tc_f = pl.pallas_call(
    _tc_f_body,
    out_shape=jax.ShapeDtypeStruct((N, D), _f32),
)


def kernel(x, edge_index, W0, b0, W1, b1):
    src = edge_index[0]
    dst = edge_index[1]
    ew, degp = _sc_edge(x, src, dst)
    hs0, dis = _tc_b(x, W0, degp.reshape(NW, N))
    part0, = _sc_agg(hs0, src, dst, ew)
    hs1 = _tc_d(part0, hs0, dis, b0, W1)
    part1, = _sc_agg(hs1, src, dst, ew)
    out = _tc_f(part1, hs1, dis, b1)
    return (out, edge_index, ew)


# speedup vs baseline: 22.1393x; 1.1295x over previous
import jax
import jax.numpy as jnp
from jax import lax
from jax.experimental import pallas as pl
from jax.experimental.pallas import tpu as pltpu
from jax.experimental.pallas import tpu_sc as plsc

N = 10000
E = 320000
D = 128

NC = 2
NS = 16
NW = NC * NS
EPW = E // NW
CH = 80
NCHUNK = EPW // CH
ZR = 16

_MESH = plsc.VectorSubcoreMesh(core_axis_name="c", subcore_axis_name="s")
_SC_PARAMS = pltpu.CompilerParams(needs_layout_passes=False)

_f32 = jnp.float32
_i32 = jnp.int32


def _sc_edge_body(x_hbm, src_hbm, dst_hbm, ew_hbm, degp_hbm,
                  sidx_all, didx_all, ew_all, deg, psum,
                  rows_s0, rows_s1, rows_d0, rows_d1,
                  gss0, gss1, gsd0, gsd1):
    cid = lax.axis_index("c")
    sid = lax.axis_index("s")
    wid = sid * NC + cid
    ebase = wid * EPW

    rows_s = (rows_s0, rows_s1)
    rows_d = (rows_d0, rows_d1)
    gss = (gss0, gss1)
    gsd = (gsd0, gsd1)

    pltpu.sync_copy(src_hbm.at[pl.ds(ebase, EPW)], sidx_all)
    pltpu.sync_copy(dst_hbm.at[pl.ds(ebase, EPW)], didx_all)

    zero16 = jnp.zeros((16,), _f32)

    def _zero(i, carry):
        deg[pl.ds(i * 16, 16)] = zero16
        return carry

    lax.fori_loop(0, N // 16, _zero, 0)

    iot = lax.iota(_i32, 16)

    def _gather(c, b):
        pltpu.async_copy(x_hbm.at[sidx_all.at[pl.ds(c * CH, CH)]],
                         rows_s[b], gss[b])
        pltpu.async_copy(x_hbm.at[didx_all.at[pl.ds(c * CH, CH)]],
                         rows_d[b], gsd[b])

    def _wait(b):
        pltpu.make_async_copy(x_hbm.at[pl.ds(0, CH)], rows_s[b],
                              gss[b]).wait()
        pltpu.make_async_copy(x_hbm.at[pl.ds(0, CH)], rows_d[b],
                              gsd[b]).wait()

    def _compute(c, b):
        cb = c * CH

        @plsc.parallel_loop(0, CH, step=1, unroll=2)
        def _edge(e):
            s = rows_s[b][e, pl.ds(0, 16)] * rows_d[b][e, pl.ds(0, 16)]
            for j in range(1, D // 16):
                s = s + (rows_s[b][e, pl.ds(j * 16, 16)] *
                         rows_d[b][e, pl.ds(j * 16, 16)])
            psum[pl.ds(e * 16, 16)] = s

        def _grp(g, gcarry):
            rowv = (g * 16 + iot) * 16
            acc = plsc.load_gather(psum, [rowv])
            for l in range(1, 16):
                acc = acc + plsc.load_gather(psum, [rowv + l])
            w = 1.0 / (1.0 + jnp.exp(-acc))
            ew_all[pl.ds(cb + g * 16, 16)] = w
            d16 = didx_all[pl.ds(cb + g * 16, 16)]
            plsc.addupdate_scatter(deg, [d16], w)
            return gcarry

        lax.fori_loop(0, CH // 16, _grp, 0)

    def _chunk(c, b, bo):
        cg = jnp.minimum(c + 1, NCHUNK - 1)
        _gather(cg, bo)
        _wait(b)
        _compute(c, b)

    _gather(0, 0)
    _chunk(0, 0, 1)

    def _super(s, carry):
        _chunk(2 * s + 1, 1, 0)
        _chunk(2 * s + 2, 0, 1)
        return carry

    lax.fori_loop(0, (NCHUNK - 1) // 2, _super, 0)

    _wait(1)

    pltpu.sync_copy(ew_all, ew_hbm.at[pl.ds(ebase, EPW)])
    pltpu.sync_copy(deg, degp_hbm.at[pl.ds(wid * N, N)])


_sc_edge = pl.kernel(
    _sc_edge_body,
    out_type=[jax.ShapeDtypeStruct((E,), _f32),
              jax.ShapeDtypeStruct((NW * N,), _f32)],
    mesh=_MESH,
    compiler_params=_SC_PARAMS,
    scratch_types=[
        pltpu.VMEM((EPW,), _i32),
        pltpu.VMEM((EPW,), _i32),
        pltpu.VMEM((EPW,), _f32),
        pltpu.VMEM((N,), _f32),
        pltpu.VMEM((CH * 16,), _f32),
        pltpu.VMEM((CH, D), _f32),
        pltpu.VMEM((CH, D), _f32),
        pltpu.VMEM((CH, D), _f32),
        pltpu.VMEM((CH, D), _f32),
        pltpu.SemaphoreType.DMA,
        pltpu.SemaphoreType.DMA,
        pltpu.SemaphoreType.DMA,
        pltpu.SemaphoreType.DMA,
    ],
)


def _sc_agg_body(h_hbm, src_hbm, dst_hbm, ew_hbm, part_hbm,
                 sidx_all,
                 rows0, rows1, rows2, ewb0, ewb1, ewb2,
                 didxb0, didxb1, didxb2, acc,
                 gs0, gs1, gs2, ss0, ss1, ss2):
    cid = lax.axis_index("c")
    sid = lax.axis_index("s")
    wid = sid * NC + cid
    ebase = wid * EPW

    rows = (rows0, rows1, rows2)
    ewb = (ewb0, ewb1, ewb2)
    didxb = (didxb0, didxb1, didxb2)
    gs = (gs0, gs1, gs2)
    ss = (ss0, ss1, ss2)

    pltpu.sync_copy(src_hbm.at[pl.ds(ebase, EPW)], sidx_all)

    rbase = sid * 624
    nblk = 39 + jnp.where(sid == NS - 1, 1, 0)

    zero16 = jnp.zeros((16,), _f32)

    def _zrow(r, carry):
        for k in range(D // 16):
            rows0[r, pl.ds(k * 16, 16)] = zero16
        return carry

    lax.fori_loop(0, ZR, _zrow, 0)

    def _zfill(t, carry):
        pltpu.sync_copy(rows0.at[pl.ds(0, ZR)],
                        acc.at[pl.ds(rbase + t * ZR, ZR)])
        return carry

    lax.fori_loop(0, nblk, _zfill, 0)
    plsc.subcore_barrier()

    def _gather(c, b):
        off = ebase + c * CH
        pltpu.async_copy(h_hbm.at[sidx_all.at[pl.ds(c * CH, CH)]],
                         rows[b], gs[b])
        pltpu.async_copy(ew_hbm.at[pl.ds(off, CH)], ewb[b], gs[b])
        pltpu.async_copy(dst_hbm.at[pl.ds(off, CH)], didxb[b], gs[b])

    def _wait_rows(b):
        pltpu.make_async_copy(h_hbm.at[pl.ds(0, CH)], rows[b], gs[b]).wait()
        pltpu.make_async_copy(ew_hbm.at[pl.ds(0, CH)], ewb[b], gs[b]).wait()
        pltpu.make_async_copy(dst_hbm.at[pl.ds(0, CH)], didxb[b],
                              gs[b]).wait()

    def _wait_scat(b):
        pltpu.make_async_copy(h_hbm.at[pl.ds(0, CH)], rows[b], ss[b]).wait()

    zero16i = jnp.zeros((16,), _i32)

    def _compute(c, b):
        @plsc.parallel_loop(0, CH, step=1, unroll=4)
        def _edge(e):
            w = plsc.load_gather(ewb[b], [zero16i + e])
            for j in range(D // 16):
                rows[b][e, pl.ds(j * 16, 16)] = (
                    rows[b][e, pl.ds(j * 16, 16)] * w)

    def _scatter(c, b):
        pltpu.async_copy(rows[b], acc.at[didxb[b]], ss[b], add=True)

    def _chunk(c, b, bn):
        _wait_scat(bn)
        cg = jnp.minimum(c + 1, NCHUNK - 1)
        _gather(cg, bn)
        _wait_rows(b)
        _compute(c, b)
        _scatter(c, b)

    _gather(0, 0)
    _gather(1, 1)
    _wait_rows(0)
    _compute(0, 0)
    _scatter(0, 0)
    _gather(2, 2)
    _wait_rows(1)
    _compute(1, 1)
    _scatter(1, 1)

    def _super(s, carry):
        _chunk(3 * s + 2, 2, 0)
        _chunk(3 * s + 3, 0, 1)
        _chunk(3 * s + 4, 1, 2)
        return carry

    lax.fori_loop(0, (NCHUNK - 2) // 3, _super, 0)

    _wait_scat(0)
    _wait_scat(1)
    _wait_rows(2)
    plsc.subcore_barrier()

    def _flush(t, carry):
        r0 = rbase + t * ZR
        pltpu.sync_copy(acc.at[pl.ds(r0, ZR)], part_hbm.at[cid, pl.ds(r0, ZR)])
        return carry

    lax.fori_loop(0, nblk, _flush, 0)


_sc_agg = pl.kernel(
    _sc_agg_body,
    out_type=[jax.ShapeDtypeStruct((NC, N, D), _f32)],
    mesh=_MESH,
    compiler_params=_SC_PARAMS,
    scratch_types=[
        pltpu.VMEM((EPW,), _i32),
        pltpu.VMEM((CH, D), _f32),
        pltpu.VMEM((CH, D), _f32),
        pltpu.VMEM((CH, D), _f32),
        pltpu.VMEM((CH,), _f32),
        pltpu.VMEM((CH,), _f32),
        pltpu.VMEM((CH,), _f32),
        pltpu.VMEM((CH,), _i32),
        pltpu.VMEM((CH,), _i32),
        pltpu.VMEM((CH,), _i32),
        pltpu.VMEM_SHARED((N, D), _f32),
        pltpu.SemaphoreType.DMA,
        pltpu.SemaphoreType.DMA,
        pltpu.SemaphoreType.DMA,
        pltpu.SemaphoreType.DMA,
        pltpu.SemaphoreType.DMA,
        pltpu.SemaphoreType.DMA,
    ],
)


def _tc_mm_body(x_ref, w_ref, h_ref):
    h_ref[...] = lax.dot_general(
        x_ref[...], w_ref[...], (((1,), (0,)), ((), ())),
        precision=lax.Precision.HIGHEST, preferred_element_type=_f32)


def _tc_s_body(h_ref, degp_ref, hs_ref, dis_ref):
    deg = 1.0 + jnp.sum(degp_ref[...], axis=0)
    dis = lax.rsqrt(deg)
    dis_ref[...] = dis
    hs_ref[...] = dis[:, None] * h_ref[...]


def _tc_d_body(part_ref, h_ref, dis_ref, b_ref, w_ref, out_ref):
    dis = dis_ref[...][:, None]
    agg = part_ref[0] + part_ref[1]
    h1 = jnp.maximum(dis * agg + dis * h_ref[...] + b_ref[...][None, :],
                     0.0)
    out_ref[...] = dis * lax.dot_general(
        h1, w_ref[...], (((1,), (0,)), ((), ())),
        precision=lax.Precision.HIGHEST, preferred_element_type=_f32)


def _tc_f_body(part_ref, g_ref, dis_ref, b_ref, out_ref):
    dis = dis_ref[...][:, None]
    agg = part_ref[0] + part_ref[1]
    out_ref[...] = dis * agg + dis * g_ref[...] + b_ref[...][None, :]


_tc_mm = pl.pallas_call(
    _tc_mm_body,
    out_shape=jax.ShapeDtypeStruct((N, D), _f32),
)

_tc_s = pl.pallas_call(
    _tc_s_body,
    out_shape=[jax.ShapeDtypeStruct((N, D), _f32),
               jax.ShapeDtypeStruct((N,), _f32)],
)

_tc_d = pl.pallas_call(
    _tc_d_body,
    out_shape=jax.ShapeDtypeStruct((N, D), _f32),
)

_tc_f = pl.pallas_call(
    _tc_f_body,
    out_shape=jax.ShapeDtypeStruct((N, D), _f32),
)


def kernel(x, edge_index, W0, b0, W1, b1):
    src = edge_index[0]
    dst = edge_index[1]
    h0 = _tc_mm(x, W0)
    ew, degp = _sc_edge(x, src, dst)
    hs0, dis = _tc_s(h0, degp.reshape(NW, N))
    part0, = _sc_agg(hs0, src, dst, ew)
    hs1 = _tc_d(part0, hs0, dis, b0, W1)
    part1, = _sc_agg(hs1, src, dst, ew)
    out = _tc_f(part1, hs1, dis, b1)
    return (out, edge_index, ew)
